# baseline (speedup 1.0000x reference)
import functools

import jax
import jax.numpy as jnp
from jax import lax
from jax.experimental import pallas as pl
from jax.experimental.pallas import tpu as pltpu
from jax.experimental.pallas import tpu_sc as plsc

_N, _E, _D = 10000, 320000, 128
_CHUNK = 64
_NW = 32
_PW = 160
_NCP = _NW * _PW
_PAD = _NCP * _CHUNK - _E
_NTRASH = 16
_ACC_N = _N + _NTRASH
_RPS = 624
_TAIL = _N - 16 * _RPS
_NBUF = 4
_NH = 10
_ZR = 32
_HG = _PW // _NH


def _make_agg():
  mesh = plsc.VectorSubcoreMesh(core_axis_name="c", subcore_axis_name="s")

  @functools.partial(
      pl.kernel,
      mesh=mesh,
      out_type=jax.ShapeDtypeStruct((2 * _N, _D), jnp.float32),
      scratch_types=[
          pltpu.VMEM((2, _HG, _CHUNK), jnp.int32),
          pltpu.VMEM((2, _HG, _CHUNK), jnp.int32),
          pltpu.VMEM((_NBUF, _CHUNK, _D), jnp.float32),
          pltpu.VMEM((_ZR, _D), jnp.float32),
          pltpu.VMEM_SHARED((_ACC_N, _D), jnp.float32),
      ] + [pltpu.SemaphoreType.DMA] * (2 * _NBUF + 4),
  )
  def agg(feat_hbm, srcs_hbm, dsts_hbm, out_hbm,
          src_v, dst_v, rows_v, zb_v, acc_sh, *sems):
    c = lax.axis_index("c")
    s = lax.axis_index("s")
    wid = c * 16 + s
    gsems = sems[:_NBUF]
    ssems = sems[_NBUF:2 * _NBUF]
    isems_s = sems[2 * _NBUF:2 * _NBUF + 2]
    isems_d = sems[2 * _NBUF + 2:]

    def gstart(p, b, t):
      pltpu.async_copy(feat_hbm.at[src_v.at[p].at[t]], rows_v.at[b],
                       gsems[b])

    def gwait(p, b, t):
      pltpu.make_async_copy(feat_hbm.at[src_v.at[p].at[t]], rows_v.at[b],
                            gsems[b]).wait()

    def sstart(p, b, t):
      pltpu.async_copy(rows_v.at[b], acc_sh.at[dst_v.at[p].at[t]], ssems[b],
                       add=True)

    def swait(p, b, t):
      pltpu.make_async_copy(rows_v.at[b], acc_sh.at[dst_v.at[p].at[t]],
                            ssems[b]).wait()

    def istart(p, h):
      off = wid * _PW + h * _HG
      pltpu.async_copy(srcs_hbm.at[pl.ds(off, _HG)], src_v.at[p], isems_s[p])
      pltpu.async_copy(dsts_hbm.at[pl.ds(off, _HG)], dst_v.at[p], isems_d[p])

    def iwait(p, h):
      off = wid * _PW + h * _HG
      pltpu.make_async_copy(srcs_hbm.at[pl.ds(off, _HG)], src_v.at[p],
                            isems_s[p]).wait()
      pltpu.make_async_copy(dsts_hbm.at[pl.ds(off, _HG)], dst_v.at[p],
                            isems_d[p]).wait()

    off0 = wid * _PW
    pltpu.sync_copy(srcs_hbm.at[pl.ds(off0, _HG)], src_v.at[0])
    pltpu.sync_copy(dsts_hbm.at[pl.ds(off0, _HG)], dst_v.at[0])
    for b in range(_NBUF):
      gstart(0, b, b)

    zvec = jnp.zeros((16,), jnp.float32)

    def zstore(i, _):
      zb_v[i // (_D // 16), pl.ds((i % (_D // 16)) * 16, 16)] = zvec
      return 0

    lax.fori_loop(0, _ZR * (_D // 16), zstore, 0)

    def zcopy(k, _):
      pltpu.sync_copy(zb_v, acc_sh.at[pl.ds(s * _RPS + k * _ZR, _ZR)])
      return 0

    lax.fori_loop(0, _RPS // _ZR, zcopy, 0)
    pltpu.sync_copy(zb_v.at[pl.ds(0, _RPS % _ZR)],
                    acc_sh.at[pl.ds(s * _RPS + (_RPS // _ZR) * _ZR,
                                    _RPS % _ZR)])

    @pl.when(s == 0)
    def _():
      pltpu.sync_copy(zb_v.at[pl.ds(0, _TAIL)],
                      acc_sh.at[pl.ds(16 * _RPS, _TAIL)])

    plsc.subcore_barrier()

    for h in range(_NH):
      p = h % 2
      if h + 1 < _NH:
        istart(1 - p, h + 1)

      def body(g, _):
        t0 = g * _NBUF
        for b in range(_NBUF):
          gwait(p, b, t0 + b)
          sstart(p, b, t0 + b)
        for b in range(_NBUF):
          swait(p, b, t0 + b)
          gstart(p, b, t0 + b + _NBUF)
        return 0

      lax.fori_loop(0, _HG // _NBUF - 1, body, 0)
      t0 = _HG - _NBUF
      for b in range(_NBUF):
        gwait(p, b, t0 + b)
        sstart(p, b, t0 + b)
      if h + 1 < _NH:
        iwait(1 - p, h + 1)
        for b in range(_NBUF):
          swait(p, b, t0 + b)
          gstart(1 - p, b, b)
      else:
        for b in range(_NBUF):
          swait(p, b, t0 + b)
    plsc.subcore_barrier()

    pltpu.sync_copy(acc_sh.at[pl.ds(s * _RPS, _RPS)],
                    out_hbm.at[pl.ds(c * _N + s * _RPS, _RPS)])

    @pl.when(s == 0)
    def _():
      pltpu.sync_copy(acc_sh.at[pl.ds(16 * _RPS, _TAIL)],
                      out_hbm.at[pl.ds(c * _N + 16 * _RPS, _TAIL)])

  return agg


_agg = _make_agg()

_BM = 2000


def _mlp_block(eps_ref, feat_ref, p0_ref, p1_ref, wa_ref, ba_ref,
               wb_ref, bb_ref, out_ref):
  h = (1.0 + eps_ref[0]) * feat_ref[...] + p0_ref[...] + p1_ref[...]
  h = jnp.maximum(
      jnp.dot(h, wa_ref[...], preferred_element_type=jnp.float32)
      + ba_ref[...], 0.0)
  h = jnp.maximum(
      jnp.dot(h, wb_ref[...], preferred_element_type=jnp.float32)
      + bb_ref[...], 0.0)
  out_ref[...] = h


def _mlp(eps, feat, parts, wa, ba, wb, bb):
  n = feat.shape[0]
  row = lambda i: (i, 0)
  p1row = lambda i: (i + _N // _BM, 0)
  whole = lambda i: (0, 0)
  return pl.pallas_call(
      _mlp_block,
      grid=(n // _BM,),
      in_specs=[
          pl.BlockSpec(memory_space=pltpu.SMEM),
          pl.BlockSpec((_BM, _D), row),
          pl.BlockSpec((_BM, _D), row),
          pl.BlockSpec((_BM, _D), p1row),
          pl.BlockSpec((_D, _D), whole),
          pl.BlockSpec((1, _D), whole),
          pl.BlockSpec((_D, _D), whole),
          pl.BlockSpec((1, _D), whole),
      ],
      out_specs=pl.BlockSpec((_BM, _D), row),
      out_shape=jax.ShapeDtypeStruct((n, _D), jnp.float32),
  )(eps, feat, parts, parts, wa, ba, wb, bb)


def _pad_edges(edge_index):
  pad = jnp.arange(_PAD, dtype=jnp.int32)
  pad_src = pad % _N
  pad_dst = _N + (pad % _NTRASH)
  srcs = jnp.concatenate([edge_index[0], pad_src]).reshape(_NCP, _CHUNK)
  dsts = jnp.concatenate([edge_index[1], pad_dst]).reshape(_NCP, _CHUNK)
  return srcs, dsts


def kernel(feat, edge_index, eps1, W1a, b1a, W1b, b1b,
           eps2, W2a, b2a, W2b, b2b):
  srcs, dsts = _pad_edges(edge_index)

  parts1 = _agg(feat, srcs, dsts)
  h1 = _mlp(eps1.reshape(1), feat, parts1,
            W1a, b1a.reshape(1, _D), W1b, b1b.reshape(1, _D))
  parts2 = _agg(h1, srcs, dsts)
  out = _mlp(eps2.reshape(1), h1, parts2,
             W2a, b2a.reshape(1, _D), W2b, b2b.reshape(1, _D))
  return out


# speedup vs baseline: 1.0108x; 1.0108x over previous
import functools

import jax
import jax.numpy as jnp
from jax import lax
from jax.experimental import pallas as pl
from jax.experimental.pallas import tpu as pltpu
from jax.experimental.pallas import tpu_sc as plsc

_N, _E, _D = 10000, 320000, 128
_CHUNK = 64
_NW = 32
_PW = 160
_NCP = _NW * _PW
_PAD = _NCP * _CHUNK - _E
_NTRASH = 16
_ACC_N = _N + _NTRASH
_RPS = 624
_TAIL = _N - 16 * _RPS
_NBUF = 4
_NH = 10
_ZR = 32
_HG = _PW // _NH


def _make_agg():
  mesh = plsc.VectorSubcoreMesh(core_axis_name="c", subcore_axis_name="s")

  @functools.partial(
      pl.kernel,
      mesh=mesh,
      out_type=jax.ShapeDtypeStruct((2 * _N, _D), jnp.float32),
      scratch_types=[
          pltpu.VMEM((2, _HG, _CHUNK), jnp.int32),
          pltpu.VMEM((2, _HG, _CHUNK), jnp.int32),
          pltpu.VMEM((_NBUF, _CHUNK, _D), jnp.float32),
          pltpu.VMEM((_ZR, _D), jnp.float32),
          pltpu.VMEM_SHARED((_ACC_N, _D), jnp.float32),
      ] + [pltpu.SemaphoreType.DMA] * (2 * _NBUF + 4),
  )
  def agg(feat_hbm, srcs_hbm, dsts_hbm, out_hbm,
          src_v, dst_v, rows_v, zb_v, acc_sh, *sems):
    c = lax.axis_index("c")
    s = lax.axis_index("s")
    wid = c * 16 + s
    gsems = sems[:_NBUF]
    ssems = sems[_NBUF:2 * _NBUF]
    isems_s = sems[2 * _NBUF:2 * _NBUF + 2]
    isems_d = sems[2 * _NBUF + 2:]

    def gstart(p, b, t):
      pltpu.async_copy(feat_hbm.at[src_v.at[p].at[t]], rows_v.at[b],
                       gsems[b])

    def gwait(p, b, t):
      pltpu.make_async_copy(feat_hbm.at[src_v.at[p].at[t]], rows_v.at[b],
                            gsems[b]).wait()

    def sstart(p, b, t):
      pltpu.async_copy(rows_v.at[b], acc_sh.at[dst_v.at[p].at[t]], ssems[b],
                       add=True)

    def swait(p, b, t):
      pltpu.make_async_copy(rows_v.at[b], acc_sh.at[dst_v.at[p].at[t]],
                            ssems[b]).wait()

    def istart(p, h):
      off = wid * _PW + h * _HG
      pltpu.async_copy(srcs_hbm.at[pl.ds(off, _HG)], src_v.at[p], isems_s[p])
      pltpu.async_copy(dsts_hbm.at[pl.ds(off, _HG)], dst_v.at[p], isems_d[p])

    def iwait(p, h):
      off = wid * _PW + h * _HG
      pltpu.make_async_copy(srcs_hbm.at[pl.ds(off, _HG)], src_v.at[p],
                            isems_s[p]).wait()
      pltpu.make_async_copy(dsts_hbm.at[pl.ds(off, _HG)], dst_v.at[p],
                            isems_d[p]).wait()

    off0 = wid * _PW
    pltpu.sync_copy(srcs_hbm.at[pl.ds(off0, _HG)], src_v.at[0])
    pltpu.sync_copy(dsts_hbm.at[pl.ds(off0, _HG)], dst_v.at[0])
    for b in range(_NBUF):
      gstart(0, b, b)

    zvec = jnp.zeros((16,), jnp.float32)

    def zstore(i, _):
      zb_v[i // (_D // 16), pl.ds((i % (_D // 16)) * 16, 16)] = zvec
      return 0

    lax.fori_loop(0, _ZR * (_D // 16), zstore, 0)

    def zcopy(k, _):
      pltpu.sync_copy(zb_v, acc_sh.at[pl.ds(s * _RPS + k * _ZR, _ZR)])
      return 0

    lax.fori_loop(0, _RPS // _ZR, zcopy, 0)
    pltpu.sync_copy(zb_v.at[pl.ds(0, _RPS % _ZR)],
                    acc_sh.at[pl.ds(s * _RPS + (_RPS // _ZR) * _ZR,
                                    _RPS % _ZR)])

    @pl.when(s == 0)
    def _():
      pltpu.sync_copy(zb_v.at[pl.ds(0, _TAIL)],
                      acc_sh.at[pl.ds(16 * _RPS, _TAIL)])

    plsc.subcore_barrier()

    for h in range(_NH):
      p = h % 2
      if h + 1 < _NH:
        istart(1 - p, h + 1)

      def body(g, _):
        t0 = g * _NBUF
        for b in range(_NBUF):
          gwait(p, b, t0 + b)
          sstart(p, b, t0 + b)
        for b in range(_NBUF):
          swait(p, b, t0 + b)
          gstart(p, b, t0 + b + _NBUF)
        return 0

      lax.fori_loop(0, _HG // _NBUF - 1, body, 0)
      t0 = _HG - _NBUF
      for b in range(_NBUF):
        gwait(p, b, t0 + b)
        sstart(p, b, t0 + b)
      if h + 1 < _NH:
        iwait(1 - p, h + 1)
        for b in range(_NBUF):
          swait(p, b, t0 + b)
          gstart(1 - p, b, b)
      else:
        for b in range(_NBUF):
          swait(p, b, t0 + b)
    plsc.subcore_barrier()

    pltpu.sync_copy(acc_sh.at[pl.ds(s * _RPS, _RPS)],
                    out_hbm.at[pl.ds(c * _N + s * _RPS, _RPS)])

    @pl.when(s == 0)
    def _():
      pltpu.sync_copy(acc_sh.at[pl.ds(16 * _RPS, _TAIL)],
                      out_hbm.at[pl.ds(c * _N + 16 * _RPS, _TAIL)])

  return agg


_agg = _make_agg()

_BM = 5000


def _mlp_block(eps_ref, feat_ref, p0_ref, p1_ref, wa_ref, ba_ref,
               wb_ref, bb_ref, out_ref):
  h = (1.0 + eps_ref[0]) * feat_ref[...] + p0_ref[...] + p1_ref[...]
  h = jnp.maximum(
      jnp.dot(h, wa_ref[...], preferred_element_type=jnp.float32)
      + ba_ref[...], 0.0)
  h = jnp.maximum(
      jnp.dot(h, wb_ref[...], preferred_element_type=jnp.float32)
      + bb_ref[...], 0.0)
  out_ref[...] = h


def _mlp(eps, feat, parts, wa, ba, wb, bb):
  n = feat.shape[0]
  row = lambda i: (i, 0)
  p1row = lambda i: (i + _N // _BM, 0)
  whole = lambda i: (0, 0)
  return pl.pallas_call(
      _mlp_block,
      grid=(n // _BM,),
      in_specs=[
          pl.BlockSpec(memory_space=pltpu.SMEM),
          pl.BlockSpec((_BM, _D), row),
          pl.BlockSpec((_BM, _D), row),
          pl.BlockSpec((_BM, _D), p1row),
          pl.BlockSpec((_D, _D), whole),
          pl.BlockSpec((1, _D), whole),
          pl.BlockSpec((_D, _D), whole),
          pl.BlockSpec((1, _D), whole),
      ],
      out_specs=pl.BlockSpec((_BM, _D), row),
      out_shape=jax.ShapeDtypeStruct((n, _D), jnp.float32),
  )(eps, feat, parts, parts, wa, ba, wb, bb)


def _pad_edges(edge_index):
  pad = jnp.arange(_PAD, dtype=jnp.int32)
  pad_src = pad % _N
  pad_dst = _N + (pad % _NTRASH)
  srcs = jnp.concatenate([edge_index[0], pad_src]).reshape(_NCP, _CHUNK)
  dsts = jnp.concatenate([edge_index[1], pad_dst]).reshape(_NCP, _CHUNK)
  return srcs, dsts


def kernel(feat, edge_index, eps1, W1a, b1a, W1b, b1b,
           eps2, W2a, b2a, W2b, b2b):
  srcs, dsts = _pad_edges(edge_index)

  parts1 = _agg(feat, srcs, dsts)
  h1 = _mlp(eps1.reshape(1), feat, parts1,
            W1a, b1a.reshape(1, _D), W1b, b1b.reshape(1, _D))
  parts2 = _agg(h1, srcs, dsts)
  out = _mlp(eps2.reshape(1), h1, parts2,
             W2a, b2a.reshape(1, _D), W2b, b2b.reshape(1, _D))
  return out
